# baseline (speedup 1.0000x reference)
import functools

import jax
import jax.numpy as jnp
from jax import lax
from jax.experimental import pallas as pl
from jax.experimental.pallas import tpu as pltpu
from jax.experimental.pallas import tpu_sc as plsc

N = 10000
D = 128
G = 128
CW = 16
NP = 10016
NW = 32
CH = 128
NCHUNK = 80
IBLK = 40
EPAD = NW * NCHUNK * CH

_mesh = plsc.VectorSubcoreMesh(core_axis_name="c", subcore_axis_name="s",
                               num_cores=2, num_subcores=16)
_sc_params = pltpu.CompilerParams(use_tc_tiling_on_sc=False)


def _sc_agg_body(y_hbm, ei_hbm, zrow_hbm, agg_out, acc_sh, idx, rows_v,
                 sem):
    c = lax.axis_index("c")
    s = lax.axis_index("s")
    wid = c * 16 + s

    rz = NP // 16
    pltpu.sync_copy(zrow_hbm.at[pl.ds(s * rz, rz)], acc_sh.at[pl.ds(s * rz, rz)])
    plsc.subcore_barrier()

    def body(j, carry):
        pltpu.sync_copy(ei_hbm.at[wid, j], idx)
        pltpu.async_copy(y_hbm.at[idx.at[0]], rows_v, sem).wait()
        pltpu.sync_copy(rows_v, acc_sh.at[idx.at[1]], add=True)
        return carry

    lax.fori_loop(0, NCHUNK, body, 0)
    plsc.subcore_barrier()

    ro = N // 16
    pltpu.sync_copy(acc_sh.at[pl.ds(s * ro, ro)],
                    agg_out.at[c, pl.ds(s * ro, ro)])


_sc_agg = functools.partial(
    pl.kernel,
    out_type=jax.ShapeDtypeStruct((2, N, D), jnp.float32),
    mesh=_mesh,
    scratch_types=[
        pltpu.VMEM_SHARED((NP, D), jnp.float32),
        pltpu.VMEM((2, CH), jnp.int32),
        pltpu.VMEM((CH, D), jnp.float32),
        pltpu.SemaphoreType.DMA,
    ],
    compiler_params=_sc_params,
)(_sc_agg_body)


def _sc_agg_cnt_body(y_hbm, ei_hbm, zrow_hbm, zcnt_hbm, ones_hbm,
                     agg_out, cnt_out, acc_sh, cnt_sh, idx, rows_v, ones_v,
                     sem):
    c = lax.axis_index("c")
    s = lax.axis_index("s")
    wid = c * 16 + s

    rz = NP // 16
    pltpu.sync_copy(zrow_hbm.at[pl.ds(s * rz, rz)], acc_sh.at[pl.ds(s * rz, rz)])
    pltpu.sync_copy(zcnt_hbm.at[pl.ds(s * rz, rz)],
                    cnt_sh.at[pl.ds(s * rz, rz)])
    pltpu.sync_copy(ones_hbm, ones_v)
    plsc.subcore_barrier()

    def body(j, carry):
        pltpu.sync_copy(ei_hbm.at[wid, j], idx)
        pltpu.async_copy(y_hbm.at[idx.at[0]], rows_v, sem).wait()
        pltpu.sync_copy(rows_v, acc_sh.at[idx.at[1]], add=True)
        pltpu.sync_copy(ones_v, cnt_sh.at[idx.at[1]], add=True)
        return carry

    lax.fori_loop(0, NCHUNK, body, 0)
    plsc.subcore_barrier()

    ro = N // 16
    pltpu.sync_copy(acc_sh.at[pl.ds(s * ro, ro)],
                    agg_out.at[c, pl.ds(s * ro, ro)])
    pltpu.sync_copy(cnt_sh.at[pl.ds(s * ro, ro)],
                    cnt_out.at[c, pl.ds(s * ro, ro)])


_sc_agg_cnt = functools.partial(
    pl.kernel,
    out_type=[jax.ShapeDtypeStruct((2, N, D), jnp.float32),
              jax.ShapeDtypeStruct((2, N, CW), jnp.float32)],
    mesh=_mesh,
    scratch_types=[
        pltpu.VMEM_SHARED((NP, D), jnp.float32),
        pltpu.VMEM_SHARED((NP, CW), jnp.float32),
        pltpu.VMEM((2, CH), jnp.int32),
        pltpu.VMEM((CH, D), jnp.float32),
        pltpu.VMEM((CH, CW), jnp.float32),
        pltpu.SemaphoreType.DMA,
    ],
    compiler_params=_sc_params,
)(_sc_agg_cnt_body)


BN = 1000
NB = N // BN
_P = lax.Precision.HIGHEST


def _cnt_col(c0, c1):
    return jnp.maximum(jnp.sum(c0[...] + c1[...], axis=1, keepdims=True)
                       * (1.0 / CW), 1.0)


def _sage_body(x, a0, a1, c0, c1, wlt, bl, wrt, hout):
    mean = (a0[...] + a1[...]) / _cnt_col(c0, c1)
    h = jnp.dot(x[...], wlt[...], preferred_element_type=jnp.float32)
    h = h + bl[...] + jnp.dot(mean, wrt[...],
                              preferred_element_type=jnp.float32)
    hout[...] = jnp.maximum(h, 0.0)


def _sage2_body(x, aa0, aa1, ab0, ab1, c0, c1, wlt, bl, wrat, wrbt, hout):
    cc = _cnt_col(c0, c1)
    meana = (aa0[...] + aa1[...]) / cc
    meanb = (ab0[...] + ab1[...]) / cc
    h = jnp.dot(x[...], wlt[...], preferred_element_type=jnp.float32)
    h = h + bl[...] + jnp.dot(meana, wrat[...],
                              preferred_element_type=jnp.float32)
    h = h + jnp.dot(meanb, wrbt[...], preferred_element_type=jnp.float32)
    hout[...] = jnp.maximum(h, 0.0)


def _tc4_body(x, a0, a1, c0, c1, wlt, bl, wrt, batch3, gsum, gcnt):
    i = pl.program_id(0)
    mean = (a0[...] + a1[...]) / _cnt_col(c0, c1)
    h4 = jnp.dot(x[...], wlt[...], preferred_element_type=jnp.float32)
    h4 = h4 + bl[...] + jnp.dot(mean, wrt[...],
                                preferred_element_type=jnp.float32)
    h4 = jnp.maximum(h4, 0.0)
    b2d = batch3[0]
    oh_gt = (lax.broadcasted_iota(jnp.int32, (G, 1), 0) == b2d
             ).astype(jnp.float32)
    ps = jnp.dot(oh_gt, h4, preferred_element_type=jnp.float32,
                 precision=_P)
    pc = jnp.sum(oh_gt, axis=1, keepdims=True)

    @pl.when(i == 0)
    def _():
        gsum[...] = ps
        gcnt[...] = pc

    @pl.when(i > 0)
    def _():
        gsum[...] = gsum[...] + ps
        gcnt[...] = gcnt[...] + pc


def _head_body(gsum, gcnt, w1t, b1, w2t, b2, w3t, b3, w4p, b4p, out):
    gm = gsum[...] / jnp.maximum(gcnt[...], 1.0)
    z = jnp.maximum(jnp.dot(gm, w1t[...],
                            preferred_element_type=jnp.float32) + b1[...], 0.0)
    z = jnp.maximum(jnp.dot(z, w2t[...],
                            preferred_element_type=jnp.float32) + b2[...], 0.0)
    z = jnp.maximum(jnp.dot(z, w3t[...],
                            preferred_element_type=jnp.float32) + b3[...], 0.0)
    out[...] = jnp.dot(z, w4p[...],
                       preferred_element_type=jnp.float32) + b4p[...]


def _rows(d):
    return pl.BlockSpec((BN, d), lambda i: (i, 0))


def _const(shape):
    return pl.BlockSpec(shape, lambda i: tuple(0 for _ in shape))


def _tc_call(body, in_specs, out_specs, out_shape):
    return pl.pallas_call(
        body,
        grid=(NB,),
        in_specs=in_specs,
        out_specs=out_specs,
        out_shape=out_shape,
    )


_f32 = jnp.float32


def _sage(x, a0, a1, c0, c1, wlt, bl, wrt, dout):
    din = x.shape[1]
    return _tc_call(
        _sage_body,
        [_rows(din), _rows(D), _rows(D), _rows(CW), _rows(CW),
         _const((din, dout)), _const((1, dout)), _const((D, dout))],
        [_rows(dout)],
        [jax.ShapeDtypeStruct((N, dout), _f32)],
    )(x, a0, a1, c0, c1, wlt, bl, wrt)[0]


def _sage2(x, aa0, aa1, ab0, ab1, c0, c1, wlt, bl, wrat, wrbt):
    return _tc_call(
        _sage2_body,
        [_rows(256)] + [_rows(D)] * 4 + [_rows(CW)] * 2 +
        [_const((256, D)), _const((1, D)), _const((D, D)), _const((D, D))],
        [_rows(D)],
        [jax.ShapeDtypeStruct((N, D), _f32)],
    )(x, aa0, aa1, ab0, ab1, c0, c1, wlt, bl, wrat, wrbt)[0]


def _combine4_pool(h, a0, a1, c0, c1, wlt, bl, wrt, batch3):
    return _tc_call(
        _tc4_body,
        [_rows(D), _rows(D), _rows(D), _rows(CW), _rows(CW),
         _const((D, D)), _const((1, D)), _const((D, D)),
         pl.BlockSpec((1, 1, BN), lambda i: (i, 0, 0))],
        [_const((G, D)), _const((G, 1))],
        [jax.ShapeDtypeStruct((G, D), _f32),
         jax.ShapeDtypeStruct((G, 1), _f32)],
    )(h, a0, a1, c0, c1, wlt, bl, wrt, batch3)


def _head(gsum, gcnt, w1t, b1, w2t, b2, w3t, b3, w4p, b4p):
    return pl.pallas_call(
        _head_body,
        out_shape=jax.ShapeDtypeStruct((G, 8), _f32),
    )(gsum, gcnt, w1t, b1, w2t, b2, w3t, b3, w4p, b4p)


def kernel(x, edge_index, batch,
           conv1_Wl, conv1_bl, conv1_Wr,
           conv2_Wl, conv2_bl, conv2_Wr,
           conv3_Wl, conv3_bl, conv3_Wr,
           conv4_Wl, conv4_bl, conv4_Wr,
           lin1_W, lin1_b, lin2_W, lin2_b,
           lin3_W, lin3_b, lin4_W, lin4_b):
    E = edge_index.shape[1]
    src = edge_index[0]
    dst = edge_index[1]
    pad = EPAD - E
    src3 = jnp.concatenate(
        [src, jnp.zeros((pad,), jnp.int32)]).reshape(NW, NCHUNK, CH)
    dst3 = jnp.concatenate(
        [dst, jnp.full((pad,), N, jnp.int32)]).reshape(NW, NCHUNK, CH)
    ei4 = jnp.stack([src3, dst3], axis=2)

    zrow = jnp.zeros((NP, D), _f32)
    zcnt = jnp.zeros((NP, CW), _f32)
    ones = jnp.ones((CH, CW), _f32)

    aggp, cntp = _sc_agg_cnt(x, ei4, zrow, zcnt, ones)
    c0, c1 = cntp[0], cntp[1]

    h1 = _sage(x, aggp[0], aggp[1], c0, c1,
               conv1_Wl.T, conv1_bl.reshape(1, -1), conv1_Wr.T, 256)

    a2a = _sc_agg(h1[:, :D], ei4, zrow)
    a2b = _sc_agg(h1[:, D:], ei4, zrow)
    h2 = _sage2(h1, a2a[0], a2a[1], a2b[0], a2b[1], c0, c1,
                conv2_Wl.T, conv2_bl.reshape(1, -1),
                conv2_Wr.T[:D], conv2_Wr.T[D:])

    a3 = _sc_agg(h2, ei4, zrow)
    h3 = _sage(h2, a3[0], a3[1], c0, c1,
               conv3_Wl.T, conv3_bl.reshape(1, -1), conv3_Wr.T, D)

    a4 = _sc_agg(h3, ei4, zrow)
    gsum, gcnt = _combine4_pool(h3, a4[0], a4[1], c0, c1,
                                conv4_Wl.T, conv4_bl.reshape(1, -1),
                                conv4_Wr.T, batch.reshape(NB, 1, BN))

    w4p = jnp.concatenate([lin4_W.T, jnp.zeros((lin4_W.shape[1], 7), _f32)],
                          axis=1)
    b4p = jnp.concatenate([lin4_b.reshape(1, 1), jnp.zeros((1, 7), _f32)],
                          axis=1)
    out = _head(gsum, gcnt,
                lin1_W.T, lin1_b.reshape(1, -1),
                lin2_W.T, lin2_b.reshape(1, -1),
                lin3_W.T, lin3_b.reshape(1, -1),
                w4p, b4p)
    return out[:, 0]


# speedup vs baseline: 1.2581x; 1.2581x over previous
import functools

import jax
import jax.numpy as jnp
from jax import lax
from jax.experimental import pallas as pl
from jax.experimental.pallas import tpu as pltpu
from jax.experimental.pallas import tpu_sc as plsc

N = 10000
D = 128
G = 128
CW = 16
NP = 10112
NW = 32
CH = 128
NCHUNK = 79
EPAD = NW * NCHUNK * CH

_mesh = plsc.VectorSubcoreMesh(core_axis_name="c", subcore_axis_name="s",
                               num_cores=2, num_subcores=16)
_sc_params = pltpu.CompilerParams(use_tc_tiling_on_sc=False)


def _sc_agg_body(with_cnt, *refs):
    if with_cnt:
        (y_hbm, src_hbm, dst_hbm, zrow_hbm, zcnt_hbm, ones_hbm,
         agg_out, cnt_out, acc_sh, cnt_sh, src_v, dst_v, rows_v, ones_v,
         sem) = refs
    else:
        (y_hbm, src_hbm, dst_hbm, zrow_hbm,
         agg_out, acc_sh, src_v, dst_v, rows_v, sem) = refs
    c = lax.axis_index("c")
    s = lax.axis_index("s")
    wid = c * 16 + s

    rz = NP // 16
    pltpu.sync_copy(zrow_hbm.at[pl.ds(s * rz, rz)], acc_sh.at[pl.ds(s * rz, rz)])
    if with_cnt:
        pltpu.sync_copy(zcnt_hbm.at[pl.ds(s * rz, rz)],
                        cnt_sh.at[pl.ds(s * rz, rz)])
        pltpu.sync_copy(ones_hbm, ones_v)
    plsc.subcore_barrier()

    def body(j, carry):
        pltpu.sync_copy(src_hbm.at[wid, j], src_v)
        pltpu.sync_copy(dst_hbm.at[wid, j], dst_v)
        pltpu.async_copy(y_hbm.at[src_v], rows_v, sem).wait()
        pltpu.sync_copy(rows_v, acc_sh.at[dst_v], add=True)
        if with_cnt:
            pltpu.sync_copy(ones_v, cnt_sh.at[dst_v], add=True)
        return carry

    lax.fori_loop(0, NCHUNK, body, 0)
    plsc.subcore_barrier()

    @pl.when(s < 15)
    def _():
        pltpu.sync_copy(acc_sh.at[pl.ds(s * 624, 624)],
                        agg_out.at[c, pl.ds(s * 624, 624)])
        if with_cnt:
            pltpu.sync_copy(cnt_sh.at[pl.ds(s * 624, 624)],
                            cnt_out.at[c, pl.ds(s * 624, 624)])

    @pl.when(s == 15)
    def _():
        pltpu.sync_copy(acc_sh.at[pl.ds(9360, 640)],
                        agg_out.at[c, pl.ds(9360, 640)])
        if with_cnt:
            pltpu.sync_copy(cnt_sh.at[pl.ds(9360, 640)],
                            cnt_out.at[c, pl.ds(9360, 640)])


_sc_agg_cnt = functools.partial(
    pl.kernel,
    out_type=[jax.ShapeDtypeStruct((2, N, D), jnp.float32),
              jax.ShapeDtypeStruct((2, N, CW), jnp.float32)],
    mesh=_mesh,
    scratch_types=[
        pltpu.VMEM_SHARED((NP, D), jnp.float32),
        pltpu.VMEM_SHARED((NP, CW), jnp.float32),
        pltpu.VMEM((CH,), jnp.int32),
        pltpu.VMEM((CH,), jnp.int32),
        pltpu.VMEM((CH, D), jnp.float32),
        pltpu.VMEM((CH, CW), jnp.float32),
        pltpu.SemaphoreType.DMA,
    ],
    compiler_params=_sc_params,
)(functools.partial(_sc_agg_body, True))


_sc_agg = functools.partial(
    pl.kernel,
    out_type=jax.ShapeDtypeStruct((2, N, D), jnp.float32),
    mesh=_mesh,
    scratch_types=[
        pltpu.VMEM_SHARED((NP, D), jnp.float32),
        pltpu.VMEM((CH,), jnp.int32),
        pltpu.VMEM((CH,), jnp.int32),
        pltpu.VMEM((CH, D), jnp.float32),
        pltpu.SemaphoreType.DMA,
    ],
    compiler_params=_sc_params,
)(functools.partial(_sc_agg_body, False))


BN = 1000
NB = N // BN
_P = lax.Precision.HIGHEST


def _cnt_col(c0, c1):
    return jnp.maximum(jnp.sum(c0[...] + c1[...], axis=1, keepdims=True)
                       * (1.0 / CW), 1.0)


def _sage_body(x, a0, a1, c0, c1, wlt, bl, wrt, hout):
    mean = (a0[...] + a1[...]) / _cnt_col(c0, c1)
    h = jnp.dot(x[...], wlt[...], preferred_element_type=jnp.float32)
    h = h + bl[...] + jnp.dot(mean, wrt[...],
                              preferred_element_type=jnp.float32)
    hout[...] = jnp.maximum(h, 0.0)


def _sage2_body(x, aa0, aa1, ab0, ab1, c0, c1, wlt, bl, wrat, wrbt, hout):
    cc = _cnt_col(c0, c1)
    meana = (aa0[...] + aa1[...]) / cc
    meanb = (ab0[...] + ab1[...]) / cc
    h = jnp.dot(x[...], wlt[...], preferred_element_type=jnp.float32)
    h = h + bl[...] + jnp.dot(meana, wrat[...],
                              preferred_element_type=jnp.float32)
    h = h + jnp.dot(meanb, wrbt[...], preferred_element_type=jnp.float32)
    hout[...] = jnp.maximum(h, 0.0)


def _tc4_body(x, a0, a1, c0, c1, wlt, bl, wrt, batch3, gsum, gcnt):
    i = pl.program_id(0)
    mean = (a0[...] + a1[...]) / _cnt_col(c0, c1)
    h4 = jnp.dot(x[...], wlt[...], preferred_element_type=jnp.float32)
    h4 = h4 + bl[...] + jnp.dot(mean, wrt[...],
                                preferred_element_type=jnp.float32)
    h4 = jnp.maximum(h4, 0.0)
    b2d = batch3[0]
    oh_gt = (lax.broadcasted_iota(jnp.int32, (G, 1), 0) == b2d
             ).astype(jnp.float32)
    ps = jnp.dot(oh_gt, h4, preferred_element_type=jnp.float32,
                 precision=_P)
    pc = jnp.sum(oh_gt, axis=1, keepdims=True)

    @pl.when(i == 0)
    def _():
        gsum[...] = ps
        gcnt[...] = pc

    @pl.when(i > 0)
    def _():
        gsum[...] = gsum[...] + ps
        gcnt[...] = gcnt[...] + pc


def _head_body(gsum, gcnt, w1t, b1, w2t, b2, w3t, b3, w4p, b4p, out):
    gm = gsum[...] / jnp.maximum(gcnt[...], 1.0)
    z = jnp.maximum(jnp.dot(gm, w1t[...],
                            preferred_element_type=jnp.float32) + b1[...], 0.0)
    z = jnp.maximum(jnp.dot(z, w2t[...],
                            preferred_element_type=jnp.float32) + b2[...], 0.0)
    z = jnp.maximum(jnp.dot(z, w3t[...],
                            preferred_element_type=jnp.float32) + b3[...], 0.0)
    out[...] = jnp.dot(z, w4p[...],
                       preferred_element_type=jnp.float32) + b4p[...]


def _rows(d):
    return pl.BlockSpec((BN, d), lambda i: (i, 0))


def _const(shape):
    return pl.BlockSpec(shape, lambda i: tuple(0 for _ in shape))


def _tc_call(body, in_specs, out_specs, out_shape):
    return pl.pallas_call(
        body,
        grid=(NB,),
        in_specs=in_specs,
        out_specs=out_specs,
        out_shape=out_shape,
    )


_f32 = jnp.float32


def _sage(x, a0, a1, c0, c1, wlt, bl, wrt, dout):
    din = x.shape[1]
    return _tc_call(
        _sage_body,
        [_rows(din), _rows(D), _rows(D), _rows(CW), _rows(CW),
         _const((din, dout)), _const((1, dout)), _const((D, dout))],
        [_rows(dout)],
        [jax.ShapeDtypeStruct((N, dout), _f32)],
    )(x, a0, a1, c0, c1, wlt, bl, wrt)[0]


def _sage2(x, aa0, aa1, ab0, ab1, c0, c1, wlt, bl, wrat, wrbt):
    return _tc_call(
        _sage2_body,
        [_rows(256)] + [_rows(D)] * 4 + [_rows(CW)] * 2 +
        [_const((256, D)), _const((1, D)), _const((D, D)), _const((D, D))],
        [_rows(D)],
        [jax.ShapeDtypeStruct((N, D), _f32)],
    )(x, aa0, aa1, ab0, ab1, c0, c1, wlt, bl, wrat, wrbt)[0]


def _combine4_pool(h, a0, a1, c0, c1, wlt, bl, wrt, batch3):
    return _tc_call(
        _tc4_body,
        [_rows(D), _rows(D), _rows(D), _rows(CW), _rows(CW),
         _const((D, D)), _const((1, D)), _const((D, D)),
         pl.BlockSpec((1, 1, BN), lambda i: (i, 0, 0))],
        [_const((G, D)), _const((G, 1))],
        [jax.ShapeDtypeStruct((G, D), _f32),
         jax.ShapeDtypeStruct((G, 1), _f32)],
    )(h, a0, a1, c0, c1, wlt, bl, wrt, batch3)


def _head(gsum, gcnt, w1t, b1, w2t, b2, w3t, b3, w4p, b4p):
    return pl.pallas_call(
        _head_body,
        out_shape=jax.ShapeDtypeStruct((G, 8), _f32),
    )(gsum, gcnt, w1t, b1, w2t, b2, w3t, b3, w4p, b4p)


def kernel(x, edge_index, batch,
           conv1_Wl, conv1_bl, conv1_Wr,
           conv2_Wl, conv2_bl, conv2_Wr,
           conv3_Wl, conv3_bl, conv3_Wr,
           conv4_Wl, conv4_bl, conv4_Wr,
           lin1_W, lin1_b, lin2_W, lin2_b,
           lin3_W, lin3_b, lin4_W, lin4_b):
    E = edge_index.shape[1]
    src = edge_index[0]
    dst = edge_index[1]
    pad = EPAD - E
    src3 = jnp.concatenate(
        [src, jnp.zeros((pad,), jnp.int32)]).reshape(NW, NCHUNK, CH)
    dst3 = jnp.concatenate(
        [dst, jnp.full((pad,), N, jnp.int32)]).reshape(NW, NCHUNK, CH)
    zrow = jnp.zeros((NP, D), _f32)
    zcnt = jnp.zeros((NP, CW), _f32)
    ones = jnp.ones((CH, CW), _f32)

    aggp, cntp = _sc_agg_cnt(x, src3, dst3, zrow, zcnt, ones)
    c0, c1 = cntp[0], cntp[1]

    h1 = _sage(x, aggp[0], aggp[1], c0, c1,
               conv1_Wl.T, conv1_bl.reshape(1, -1), conv1_Wr.T, 256)

    a2a = _sc_agg(h1[:, :D], src3, dst3, zrow)
    a2b = _sc_agg(h1[:, D:], src3, dst3, zrow)
    h2 = _sage2(h1, a2a[0], a2a[1], a2b[0], a2b[1], c0, c1,
                conv2_Wl.T, conv2_bl.reshape(1, -1),
                conv2_Wr.T[:D], conv2_Wr.T[D:])

    a3 = _sc_agg(h2, src3, dst3, zrow)
    h3 = _sage(h2, a3[0], a3[1], c0, c1,
               conv3_Wl.T, conv3_bl.reshape(1, -1), conv3_Wr.T, D)

    a4 = _sc_agg(h3, src3, dst3, zrow)
    gsum, gcnt = _combine4_pool(h3, a4[0], a4[1], c0, c1,
                                conv4_Wl.T, conv4_bl.reshape(1, -1),
                                conv4_Wr.T, batch.reshape(NB, 1, BN))

    w4p = jnp.concatenate([lin4_W.T, jnp.zeros((lin4_W.shape[1], 7), _f32)],
                          axis=1)
    b4p = jnp.concatenate([lin4_b.reshape(1, 1), jnp.zeros((1, 7), _f32)],
                          axis=1)
    out = _head(gsum, gcnt,
                lin1_W.T, lin1_b.reshape(1, -1),
                lin2_W.T, lin2_b.reshape(1, -1),
                lin3_W.T, lin3_b.reshape(1, -1),
                w4p, b4p)
    return out[:, 0]
